# baseline (speedup 1.0000x reference)
import jax
import jax.numpy as jnp
from jax.experimental import pallas as pl
from jax.experimental.pallas import tpu as pltpu

NUM_EMBED = 1024
EMBED_DIM = 256
BETA = 0.25

N_BATCH = 16
N_PIX = 1024
N_ROWS = N_BATCH * N_PIX


def _vq_tc_body(z_ref, w_ref, wn_ref, enc_ref, idx_ref, zq_ref, loss_ref,
                perp_ref, cnt_ref, acc_ref):
    step = pl.program_id(0)

    @pl.when(step == 0)
    def _():
        cnt_ref[...] = jnp.zeros_like(cnt_ref)
        acc_ref[0] = 0.0

    zbt = z_ref[0]
    w = w_ref[...]
    wn = wn_ref[...]

    zb = zbt.T
    zsq = jnp.sum(zb * zb, axis=1, keepdims=True)
    nz = zb / jnp.maximum(jnp.sqrt(zsq), 1e-12)

    d = jax.lax.dot_general(nz, wn, (((1,), (1,)), ((), ())),
                            precision=jax.lax.Precision.DEFAULT,
                            preferred_element_type=jnp.float32)

    dmax = jnp.max(d, axis=1, keepdims=True)
    iota = jax.lax.broadcasted_iota(jnp.int32, d.shape, 1)
    idx = jnp.max(jnp.where(d == dmax, iota, -1), axis=1, keepdims=True)

    enc = (iota == idx).astype(jnp.float32)
    enc_ref[...] = enc
    idx_ref[...] = idx
    cnt_ref[...] += jnp.sum(enc, axis=0, keepdims=True)

    iota_c = jax.lax.broadcasted_iota(jnp.int32, (NUM_EMBED, N_PIX), 0)
    enc_t = (iota_c == idx.T).astype(jnp.float32)
    zq = jax.lax.dot_general(w, enc_t, (((0,), (0,)), ((), ())),
                             precision=jax.lax.Precision.DEFAULT,
                             preferred_element_type=jnp.float32)
    zq_ref[0] = zq

    diff = zq - zbt
    bsum = jnp.sum(diff * diff)
    total = acc_ref[0] + bsum
    acc_ref[0] = total

    @pl.when(step == N_BATCH - 1)
    def _():
        loss_ref[0] = (1.0 + BETA) * total / (N_ROWS * EMBED_DIM)
        p = cnt_ref[...] / N_ROWS
        perp_ref[0] = jnp.exp(-jnp.sum(p * jnp.log(p + 1e-10)))


@jax.jit
def _vq_tc(zr, w, wn):
    out_shapes = (
        jax.ShapeDtypeStruct((N_ROWS, NUM_EMBED), jnp.float32),
        jax.ShapeDtypeStruct((N_ROWS, 1), jnp.int32),
        jax.ShapeDtypeStruct((N_BATCH, EMBED_DIM, N_PIX), jnp.float32),
        jax.ShapeDtypeStruct((1,), jnp.float32),
        jax.ShapeDtypeStruct((1,), jnp.float32),
    )
    return pl.pallas_call(
        _vq_tc_body,
        grid=(N_BATCH,),
        in_specs=[
            pl.BlockSpec((1, EMBED_DIM, N_PIX), lambda i: (i, 0, 0)),
            pl.BlockSpec((NUM_EMBED, EMBED_DIM), lambda i: (0, 0)),
            pl.BlockSpec((NUM_EMBED, EMBED_DIM), lambda i: (0, 0)),
        ],
        out_specs=(
            pl.BlockSpec((N_PIX, NUM_EMBED), lambda i: (i, 0)),
            pl.BlockSpec((N_PIX, 1), lambda i: (i, 0)),
            pl.BlockSpec((1, EMBED_DIM, N_PIX), lambda i: (i, 0, 0)),
            pl.BlockSpec(memory_space=pltpu.SMEM),
            pl.BlockSpec(memory_space=pltpu.SMEM),
        ),
        out_shape=out_shapes,
        scratch_shapes=[
            pltpu.VMEM((1, NUM_EMBED), jnp.float32),
            pltpu.SMEM((1,), jnp.float32),
        ],
    )(zr, w, wn)


def kernel(z, W, training):
    zr = z.reshape(N_BATCH, EMBED_DIM, N_PIX)

    wn = W / jnp.maximum(
        jnp.sqrt(jnp.sum(W * W, axis=1, keepdims=True)), 1e-12)

    enc, idx2d, zq, loss, perp = _vq_tc(zr, W, wn)

    z_q_out = zq.reshape(N_BATCH, EMBED_DIM, 32, 32)
    encoding_indices = idx2d.reshape(N_ROWS)
    return (loss[0], z_q_out, perp[0], enc, encoding_indices)


# speedup vs baseline: 1.0611x; 1.0611x over previous
import jax
import jax.numpy as jnp
from jax.experimental import pallas as pl
from jax.experimental.pallas import tpu as pltpu

NUM_EMBED = 1024
EMBED_DIM = 256
BETA = 0.25

N_BATCH = 16
N_PIX = 1024
N_ROWS = N_BATCH * N_PIX


def _vq_tc_body(z_ref, s_ref, wn_ref, wt_ref, enc_ref, idx_ref, zq_ref,
                loss_ref, perp_ref, cnt_ref, acc_ref):
    step = pl.program_id(0)

    @pl.when(step == 0)
    def _():
        cnt_ref[...] = jnp.zeros_like(cnt_ref)
        acc_ref[0] = 0.0

    zbt = z_ref[0]
    srow = s_ref[0]
    wn = wn_ref[...]
    wt = wt_ref[...]

    nzt = zbt / jnp.maximum(jnp.sqrt(srow), 1e-12)

    dt = jax.lax.dot_general(wn, nzt, (((1,), (0,)), ((), ())),
                             precision=jax.lax.Precision.DEFAULT,
                             preferred_element_type=jnp.float32)

    dmax = jnp.max(dt, axis=0, keepdims=True)
    iota_c = jax.lax.broadcasted_iota(jnp.int32, dt.shape, 0)
    idx_row = jnp.max(jnp.where(dt == dmax, iota_c, -1), axis=0,
                      keepdims=True)

    enc_t = (iota_c == idx_row).astype(jnp.float32)
    idx_ref[0] = idx_row
    cnt_ref[...] += jnp.sum(enc_t, axis=1, keepdims=True)

    idx_col = idx_row.T
    iota_k = jax.lax.broadcasted_iota(jnp.int32, (N_PIX, NUM_EMBED), 1)
    enc_ref[...] = (iota_k == idx_col).astype(jnp.float32)

    zq = jax.lax.dot_general(wt, enc_t, (((1,), (0,)), ((), ())),
                             precision=jax.lax.Precision.DEFAULT,
                             preferred_element_type=jnp.float32)
    zq_ref[0] = zq

    diff = zq - zbt
    bsum = jnp.sum(diff * diff)
    total = acc_ref[0] + bsum
    acc_ref[0] = total

    @pl.when(step == N_BATCH - 1)
    def _():
        loss_ref[0] = (1.0 + BETA) * total / (N_ROWS * EMBED_DIM)
        p = cnt_ref[...] / N_ROWS
        perp_ref[0] = jnp.exp(-jnp.sum(p * jnp.log(p + 1e-10)))


@jax.jit
def _vq_tc(zr, s3, wn, wt):
    out_shapes = (
        jax.ShapeDtypeStruct((N_ROWS, NUM_EMBED), jnp.float32),
        jax.ShapeDtypeStruct((N_BATCH, 1, N_PIX), jnp.int32),
        jax.ShapeDtypeStruct((N_BATCH, EMBED_DIM, N_PIX), jnp.float32),
        jax.ShapeDtypeStruct((1,), jnp.float32),
        jax.ShapeDtypeStruct((1,), jnp.float32),
    )
    return pl.pallas_call(
        _vq_tc_body,
        grid=(N_BATCH,),
        in_specs=[
            pl.BlockSpec((1, EMBED_DIM, N_PIX), lambda i: (i, 0, 0)),
            pl.BlockSpec((1, 1, N_PIX), lambda i: (i, 0, 0)),
            pl.BlockSpec((NUM_EMBED, EMBED_DIM), lambda i: (0, 0)),
            pl.BlockSpec((EMBED_DIM, NUM_EMBED), lambda i: (0, 0)),
        ],
        out_specs=(
            pl.BlockSpec((N_PIX, NUM_EMBED), lambda i: (i, 0)),
            pl.BlockSpec((1, 1, N_PIX), lambda i: (i, 0, 0)),
            pl.BlockSpec((1, EMBED_DIM, N_PIX), lambda i: (i, 0, 0)),
            pl.BlockSpec(memory_space=pltpu.SMEM),
            pl.BlockSpec(memory_space=pltpu.SMEM),
        ),
        out_shape=out_shapes,
        scratch_shapes=[
            pltpu.VMEM((NUM_EMBED, 1), jnp.float32),
            pltpu.SMEM((1,), jnp.float32),
        ],
    )(zr, s3, wn, wt)


def kernel(z, W, training):
    zr = z.reshape(N_BATCH, EMBED_DIM, N_PIX)

    zf = jnp.transpose(z, (0, 2, 3, 1)).reshape(-1, EMBED_DIM)
    s3 = jnp.sum(zf * zf, axis=1).reshape(N_BATCH, 1, N_PIX)

    wn = W / jnp.maximum(
        jnp.sqrt(jnp.sum(W * W, axis=1, keepdims=True)), 1e-12)
    wt = W.T

    enc, idx3d, zq, loss, perp = _vq_tc(zr, s3, wn, wt)

    z_q_out = zq.reshape(N_BATCH, EMBED_DIM, 32, 32)
    encoding_indices = idx3d.reshape(N_ROWS)
    return (loss[0], z_q_out, perp[0], enc, encoding_indices)


# speedup vs baseline: 1.0668x; 1.0054x over previous
import jax
import jax.numpy as jnp
from jax.experimental import pallas as pl
from jax.experimental.pallas import tpu as pltpu

NUM_EMBED = 1024
EMBED_DIM = 256
BETA = 0.25

N_BATCH = 16
N_PIX = 1024
N_ROWS = N_BATCH * N_PIX
BPB = 2
N_STEPS = N_BATCH // BPB


def _vq_tc_body(z_ref, s_ref, wn_ref, wt_ref, enc_ref, idx_ref, zq_ref,
                loss_ref, perp_ref, cnt_ref, acc_ref):
    step = pl.program_id(0)

    @pl.when(step == 0)
    def _():
        cnt_ref[...] = jnp.zeros_like(cnt_ref)
        acc_ref[0] = 0.0

    wn = wn_ref[...]
    wt = wt_ref[...]

    bsum = 0.0
    cnt_upd = jnp.zeros((NUM_EMBED, 1), jnp.float32)
    for bb in range(BPB):
        zbt = z_ref[bb]
        srow = s_ref[bb]

        nzt = zbt / jnp.maximum(jnp.sqrt(srow), 1e-12)

        dt = jax.lax.dot_general(wn, nzt, (((1,), (0,)), ((), ())),
                                 precision=jax.lax.Precision.DEFAULT,
                                 preferred_element_type=jnp.float32)

        dmax = jnp.max(dt, axis=0, keepdims=True)
        iota_c = jax.lax.broadcasted_iota(jnp.int32, dt.shape, 0)
        idx_row = jnp.max(jnp.where(dt == dmax, iota_c, -1), axis=0,
                          keepdims=True)

        enc_t = (iota_c == idx_row).astype(jnp.float32)
        idx_ref[bb] = idx_row
        cnt_upd += jnp.sum(enc_t, axis=1, keepdims=True)

        idx_col = idx_row.T
        iota_k = jax.lax.broadcasted_iota(jnp.int32, (N_PIX, NUM_EMBED), 1)
        enc_ref[pl.ds(bb * N_PIX, N_PIX), :] = (
            (iota_k == idx_col).astype(jnp.float32))

        zq = jax.lax.dot_general(wt, enc_t, (((1,), (0,)), ((), ())),
                                 precision=jax.lax.Precision.DEFAULT,
                                 preferred_element_type=jnp.float32)
        zq_ref[bb] = zq

        diff = zq - zbt
        bsum = bsum + jnp.sum(diff * diff)

    cnt_ref[...] += cnt_upd
    total = acc_ref[0] + bsum
    acc_ref[0] = total

    @pl.when(step == N_STEPS - 1)
    def _():
        loss_ref[0] = (1.0 + BETA) * total / (N_ROWS * EMBED_DIM)
        p = cnt_ref[...] / N_ROWS
        perp_ref[0] = jnp.exp(-jnp.sum(p * jnp.log(p + 1e-10)))


@jax.jit
def _vq_tc(zr, s3, wn, wt):
    out_shapes = (
        jax.ShapeDtypeStruct((N_ROWS, NUM_EMBED), jnp.float32),
        jax.ShapeDtypeStruct((N_BATCH, 1, N_PIX), jnp.int32),
        jax.ShapeDtypeStruct((N_BATCH, EMBED_DIM, N_PIX), jnp.float32),
        jax.ShapeDtypeStruct((1,), jnp.float32),
        jax.ShapeDtypeStruct((1,), jnp.float32),
    )
    return pl.pallas_call(
        _vq_tc_body,
        grid=(N_STEPS,),
        in_specs=[
            pl.BlockSpec((BPB, EMBED_DIM, N_PIX), lambda i: (i, 0, 0)),
            pl.BlockSpec((BPB, 1, N_PIX), lambda i: (i, 0, 0)),
            pl.BlockSpec((NUM_EMBED, EMBED_DIM), lambda i: (0, 0)),
            pl.BlockSpec((EMBED_DIM, NUM_EMBED), lambda i: (0, 0)),
        ],
        out_specs=(
            pl.BlockSpec((BPB * N_PIX, NUM_EMBED), lambda i: (i, 0)),
            pl.BlockSpec((BPB, 1, N_PIX), lambda i: (i, 0, 0)),
            pl.BlockSpec((BPB, EMBED_DIM, N_PIX), lambda i: (i, 0, 0)),
            pl.BlockSpec(memory_space=pltpu.SMEM),
            pl.BlockSpec(memory_space=pltpu.SMEM),
        ),
        out_shape=out_shapes,
        scratch_shapes=[
            pltpu.VMEM((NUM_EMBED, 1), jnp.float32),
            pltpu.SMEM((1,), jnp.float32),
        ],
    )(zr, s3, wn, wt)


def kernel(z, W, training):
    zr = z.reshape(N_BATCH, EMBED_DIM, N_PIX)

    zf = jnp.transpose(z, (0, 2, 3, 1)).reshape(-1, EMBED_DIM)
    s3 = jnp.sum(zf * zf, axis=1).reshape(N_BATCH, 1, N_PIX)

    wn = W / jnp.maximum(
        jnp.sqrt(jnp.sum(W * W, axis=1, keepdims=True)), 1e-12)
    wt = W.T

    enc, idx3d, zq, loss, perp = _vq_tc(zr, s3, wn, wt)

    z_q_out = zq.reshape(N_BATCH, EMBED_DIM, 32, 32)
    encoding_indices = idx3d.reshape(N_ROWS)
    return (loss[0], z_q_out, perp[0], enc, encoding_indices)
